# baseline (speedup 1.0000x reference)
import functools

import jax
import jax.numpy as jnp
from jax import lax
from jax.experimental import pallas as pl
from jax.experimental.pallas import tpu as pltpu
from jax.experimental.pallas import tpu_sc as plsc

_NUM_ROWS = 4096 * 200
_DIM = 32
_VOCAB = 1000000
_FULL_TILES = _VOCAB // 128
_REM = _VOCAB - _FULL_TILES * 128

_INFO = plsc.get_sparse_core_info()
_NC = _INFO.num_cores
_NS = _INFO.num_subcores
_NW = _NC * _NS
_ROWS_PER_W = _NUM_ROWS // _NW
_CHUNK = 1600
_NCHUNK = _ROWS_PER_W // _CHUNK

_TPW = _FULL_TILES // _NW
_TXTRA = _FULL_TILES - _TPW * _NW

_mesh = plsc.VectorSubcoreMesh(core_axis_name="c", subcore_axis_name="s")


@functools.partial(
    pl.kernel,
    mesh=_mesh,
    out_type=jax.ShapeDtypeStruct((_VOCAB * _DIM,), jnp.float32),
    scratch_types=[
        pltpu.VMEM((2, 4, 8, 129), jnp.float32),
        pltpu.VMEM((4096,), jnp.float32),
        pltpu.VMEM((4096,), jnp.float32),
        pltpu.SemaphoreType.DMA,
        pltpu.SemaphoreType.DMA,
        pltpu.SemaphoreType.DMA,
        pltpu.SemaphoreType.DMA,
    ],
    compiler_params=pltpu.CompilerParams(use_tc_tiling_on_sc=True,
                                         needs_layout_passes=False),
)
def _format_kernel(wt_hbm, out_hbm, in_v, tr0, tr1, s_in0, s_in1,
                   s_out0, s_out1):
    wid = lax.axis_index("s") * _NC + lax.axis_index("c")
    c0 = wid * _TPW
    s_in = (s_in0, s_in1)
    s_out = (s_out0, s_out1)
    tr = (tr0, tr1)

    iota = lax.iota(jnp.int32, 16)
    r_lo = iota >> 3
    s_lo = iota & 7
    r_hi = r_lo + 2

    def start_in(c, b):
        for r in range(4):
            pltpu.async_copy(
                wt_hbm.at[pl.ds(8 * r, 8), pl.ds(c * 128, 128)],
                in_v.at[b, r, :, pl.ds(0, 128)], s_in[b])

    def wait_in(b):
        for r in range(4):
            pltpu.make_async_copy(
                wt_hbm.at[pl.ds(0, 8), pl.ds(0, 128)],
                in_v.at[b, r, :, pl.ds(0, 128)], s_in[b]).wait()

    def start_out(c, b):
        pltpu.async_copy(tr[b], out_hbm.at[pl.ds(c * 4096, 4096)], s_out[b])

    def wait_out(b):
        pltpu.make_async_copy(tr[b], out_hbm.at[pl.ds(0, 4096)],
                              s_out[b]).wait()

    def transpose_chunk(b):
        src_ref = in_v.at[b]
        dst = tr[b]

        def body(v, carry):
            vv = jnp.full((16,), v, jnp.int32)
            lo = plsc.load_gather(src_ref, [r_lo, s_lo, vv])
            hi = plsc.load_gather(src_ref, [r_hi, s_lo, vv])
            dst[pl.ds(v * 32, 16)] = lo
            dst[pl.ds(v * 32 + 16, 16)] = hi
            return carry

        lax.fori_loop(0, 128, body, 0, unroll=8)

    start_in(c0, 0)

    def pair_body(k, carry):
        for b in (0, 1):
            i = 2 * k + b
            c = c0 + i
            start_in(c + 1, 1 - b)
            wait_in(b)

            @pl.when(k >= 1)
            def _wout():
                wait_out(b)

            transpose_chunk(b)
            start_out(c, b)
        return carry

    lax.fori_loop(0, _TPW // 2, pair_body, 0)
    wait_out(0)
    wait_out(1)
    wait_in(0)

    @pl.when(wid < _TXTRA)
    def _extra():
        c = _FULL_TILES - _TXTRA + wid
        start_in(c, 0)
        wait_in(0)
        transpose_chunk(0)
        start_out(c, 0)
        wait_out(0)


@functools.partial(
    pl.kernel,
    mesh=_mesh,
    out_type=jax.ShapeDtypeStruct((_NUM_ROWS, _DIM), jnp.float32),
    scratch_types=[
        pltpu.VMEM((2, _CHUNK), jnp.int32),
        pltpu.VMEM((2, _CHUNK, _DIM), jnp.float32),
        pltpu.SemaphoreType.DMA,
        pltpu.SemaphoreType.DMA,
        pltpu.SemaphoreType.DMA,
        pltpu.SemaphoreType.DMA,
        pltpu.SemaphoreType.DMA,
        pltpu.SemaphoreType.DMA,
    ],
    compiler_params=pltpu.CompilerParams(use_tc_tiling_on_sc=False),
)
def _gather_kernel(idx_hbm, table_hbm, out_hbm, idx_v, rows_v,
                   s_i0, s_i1, s_g0, s_g1, s_o0, s_o1):
    wid = lax.axis_index("s") * _NC + lax.axis_index("c")
    base = wid * _ROWS_PER_W
    s_idx = (s_i0, s_i1)
    s_gat = (s_g0, s_g1)
    s_out = (s_o0, s_o1)

    def start_idx(i):
        b = i % 2
        return pltpu.async_copy(
            idx_hbm.at[pl.ds(base + i * _CHUNK, _CHUNK)], idx_v.at[b], s_idx[b])

    def start_gather(i):
        b = i % 2
        return pltpu.async_copy(table_hbm.at[idx_v.at[b]], rows_v.at[b], s_gat[b])

    def start_out(i):
        b = i % 2
        return pltpu.async_copy(
            rows_v.at[b], out_hbm.at[pl.ds(base + i * _CHUNK, _CHUNK)], s_out[b])

    idx_h = [None] * _NCHUNK
    gat_h = [None] * _NCHUNK
    out_h = [None] * _NCHUNK

    idx_h[0] = start_idx(0)
    idx_h[1] = start_idx(1)
    for i in range(_NCHUNK):
        b = i % 2
        idx_h[i].wait()
        if i >= 2:
            out_h[i - 2].wait()
        gat_h[i] = start_gather(i)
        if i >= 1:
            gat_h[i - 1].wait()
            out_h[i - 1] = start_out(i - 1)
            if i + 1 < _NCHUNK:
                idx_h[i + 1] = start_idx(i + 1)
    gat_h[_NCHUNK - 1].wait()
    out_h[_NCHUNK - 1] = start_out(_NCHUNK - 1)
    out_h[_NCHUNK - 2].wait()
    out_h[_NCHUNK - 1].wait()


def kernel(input_, weight):
    idx = input_.reshape(-1).astype(jnp.int32)
    table_flat = _format_kernel(weight.T)
    tail = weight[_FULL_TILES * 128:, :].reshape(-1)
    table_flat = lax.dynamic_update_slice(
        table_flat, tail, (_FULL_TILES * 128 * _DIM,))
    table = table_flat.reshape(_VOCAB, _DIM)
    out = _gather_kernel(idx, table)
    return out.reshape(input_.shape + (weight.shape[1],))


# speedup vs baseline: 1.0001x; 1.0001x over previous
import functools

import jax
import jax.numpy as jnp
from jax import lax
from jax.experimental import pallas as pl
from jax.experimental.pallas import tpu as pltpu
from jax.experimental.pallas import tpu_sc as plsc

_NUM_ROWS = 4096 * 200
_DIM = 32
_VOCAB = 1000000
_FULL_TILES = _VOCAB // 128
_REM = _VOCAB - _FULL_TILES * 128

_INFO = plsc.get_sparse_core_info()
_NC = _INFO.num_cores
_NS = _INFO.num_subcores
_NW = _NC * _NS
_ROWS_PER_W = _NUM_ROWS // _NW
_CHUNK = 1600
_NCHUNK = _ROWS_PER_W // _CHUNK

_TPW = _FULL_TILES // _NW
_TXTRA = _FULL_TILES - _TPW * _NW

_mesh = plsc.VectorSubcoreMesh(core_axis_name="c", subcore_axis_name="s")


@functools.partial(
    pl.kernel,
    mesh=_mesh,
    out_type=jax.ShapeDtypeStruct((_VOCAB * _DIM,), jnp.float32),
    scratch_types=[
        pltpu.VMEM((4, 32, 129), jnp.float32),
        pltpu.VMEM((4096,), jnp.float32),
        pltpu.VMEM((4096,), jnp.float32),
        pltpu.VMEM((4096,), jnp.float32),
        pltpu.VMEM((4096,), jnp.float32),
        pltpu.SemaphoreType.DMA,
        pltpu.SemaphoreType.DMA,
        pltpu.SemaphoreType.DMA,
        pltpu.SemaphoreType.DMA,
        pltpu.SemaphoreType.DMA,
        pltpu.SemaphoreType.DMA,
        pltpu.SemaphoreType.DMA,
        pltpu.SemaphoreType.DMA,
    ],
    compiler_params=pltpu.CompilerParams(use_tc_tiling_on_sc=True,
                                         needs_layout_passes=False),
)
def _format_kernel(wt_hbm, out_hbm, in_v, tr0, tr1, tr2, tr3,
                   s_i0, s_i1, s_i2, s_i3, s_o0, s_o1, s_o2, s_o3):
    wid = lax.axis_index("s") * _NC + lax.axis_index("c")
    c0 = wid * _TPW
    s_in = (s_i0, s_i1, s_i2, s_i3)
    s_out = (s_o0, s_o1, s_o2, s_o3)
    tr = (tr0, tr1, tr2, tr3)

    iota = lax.iota(jnp.int32, 16)
    d_lo = iota
    d_hi = iota + 16

    def start_in(c, b):
        pltpu.async_copy(wt_hbm.at[:, pl.ds(c * 128, 128)],
                         in_v.at[b, :, pl.ds(0, 128)], s_in[b])

    def wait_in(b):
        pltpu.make_async_copy(wt_hbm.at[:, pl.ds(0, 128)],
                              in_v.at[b, :, pl.ds(0, 128)], s_in[b]).wait()

    def start_out(c, b):
        pltpu.async_copy(tr[b], out_hbm.at[pl.ds(c * 4096, 4096)], s_out[b])

    def wait_out(b):
        pltpu.make_async_copy(tr[b], out_hbm.at[pl.ds(0, 4096)],
                              s_out[b]).wait()

    def transpose_chunk(b):
        src_ref = in_v.at[b]
        dst = tr[b]

        def body(v, carry):
            vv = jnp.full((16,), v, jnp.int32)
            lo = plsc.load_gather(src_ref, [d_lo, vv])
            hi = plsc.load_gather(src_ref, [d_hi, vv])
            dst[pl.ds(v * 32, 16)] = lo
            dst[pl.ds(v * 32 + 16, 16)] = hi
            return carry

        lax.fori_loop(0, 128, body, 0, unroll=8)

    start_in(c0, 0)
    start_in(c0 + 1, 1)
    start_in(c0 + 2, 2)

    def quad_body(k, carry):
        for b in (0, 1, 2, 3):
            i = 4 * k + b
            c = c0 + i
            start_in(c + 3, (b + 3) % 4)
            wait_in(b)

            @pl.when(k >= 1)
            def _wout():
                wait_out(b)

            transpose_chunk(b)
            start_out(c, b)
        return carry

    lax.fori_loop(0, _TPW // 4, quad_body, 0)
    for b in range(4):
        wait_out(b)
    for b in range(3):
        wait_in(b)

    @pl.when(wid < _TXTRA)
    def _extra():
        c = _FULL_TILES - _TXTRA + wid
        start_in(c, 3)
        wait_in(3)
        transpose_chunk(3)
        start_out(c, 3)
        wait_out(3)


@functools.partial(
    pl.kernel,
    mesh=_mesh,
    out_type=jax.ShapeDtypeStruct((_NUM_ROWS, _DIM), jnp.float32),
    scratch_types=[
        pltpu.VMEM((2, _CHUNK), jnp.int32),
        pltpu.VMEM((2, _CHUNK, _DIM), jnp.float32),
        pltpu.SemaphoreType.DMA,
        pltpu.SemaphoreType.DMA,
        pltpu.SemaphoreType.DMA,
        pltpu.SemaphoreType.DMA,
        pltpu.SemaphoreType.DMA,
        pltpu.SemaphoreType.DMA,
    ],
    compiler_params=pltpu.CompilerParams(use_tc_tiling_on_sc=False),
)
def _gather_kernel(idx_hbm, table_hbm, out_hbm, idx_v, rows_v,
                   s_i0, s_i1, s_g0, s_g1, s_o0, s_o1):
    wid = lax.axis_index("s") * _NC + lax.axis_index("c")
    base = wid * _ROWS_PER_W
    s_idx = (s_i0, s_i1)
    s_gat = (s_g0, s_g1)
    s_out = (s_o0, s_o1)

    def start_idx(i):
        b = i % 2
        return pltpu.async_copy(
            idx_hbm.at[pl.ds(base + i * _CHUNK, _CHUNK)], idx_v.at[b], s_idx[b])

    def start_gather(i):
        b = i % 2
        return pltpu.async_copy(table_hbm.at[idx_v.at[b]], rows_v.at[b], s_gat[b])

    def start_out(i):
        b = i % 2
        return pltpu.async_copy(
            rows_v.at[b], out_hbm.at[pl.ds(base + i * _CHUNK, _CHUNK)], s_out[b])

    idx_h = [None] * _NCHUNK
    gat_h = [None] * _NCHUNK
    out_h = [None] * _NCHUNK

    idx_h[0] = start_idx(0)
    idx_h[1] = start_idx(1)
    for i in range(_NCHUNK):
        b = i % 2
        idx_h[i].wait()
        if i >= 2:
            out_h[i - 2].wait()
        gat_h[i] = start_gather(i)
        if i >= 1:
            gat_h[i - 1].wait()
            out_h[i - 1] = start_out(i - 1)
            if i + 1 < _NCHUNK:
                idx_h[i + 1] = start_idx(i + 1)
    gat_h[_NCHUNK - 1].wait()
    out_h[_NCHUNK - 1] = start_out(_NCHUNK - 1)
    out_h[_NCHUNK - 2].wait()
    out_h[_NCHUNK - 1].wait()


def kernel(input_, weight):
    idx = input_.reshape(-1).astype(jnp.int32)
    table_flat = _format_kernel(weight.T)
    tail = weight[_FULL_TILES * 128:, :].reshape(-1)
    table_flat = lax.dynamic_update_slice(
        table_flat, tail, (_FULL_TILES * 128 * _DIM,))
    table = table_flat.reshape(_VOCAB, _DIM)
    out = _gather_kernel(idx, table)
    return out.reshape(input_.shape + (weight.shape[1],))


# speedup vs baseline: 1.1803x; 1.1802x over previous
import functools

import jax
import jax.numpy as jnp
from jax import lax
from jax.experimental import pallas as pl
from jax.experimental.pallas import tpu as pltpu
from jax.experimental.pallas import tpu_sc as plsc

_NUM_ROWS = 4096 * 200
_DIM = 32
_VOCAB = 1000000
_FULL_TILES = _VOCAB // 128
_REM = _VOCAB - _FULL_TILES * 128

_INFO = plsc.get_sparse_core_info()
_NC = _INFO.num_cores
_NS = _INFO.num_subcores
_NW = _NC * _NS
_ROWS_PER_W = _NUM_ROWS // _NW
_CHUNK = 1600
_NCHUNK = _ROWS_PER_W // _CHUNK

_TPW = _FULL_TILES // _NW
_TXTRA = _FULL_TILES - _TPW * _NW

_mesh = plsc.VectorSubcoreMesh(core_axis_name="c", subcore_axis_name="s")


@functools.partial(
    pl.kernel,
    mesh=_mesh,
    out_type=jax.ShapeDtypeStruct((_VOCAB * _DIM,), jnp.float32),
    scratch_types=[
        pltpu.VMEM((4, 32, 129), jnp.float32),
        pltpu.VMEM((4096,), jnp.float32),
        pltpu.VMEM((4096,), jnp.float32),
        pltpu.VMEM((4096,), jnp.float32),
        pltpu.VMEM((4096,), jnp.float32),
        pltpu.SemaphoreType.DMA,
        pltpu.SemaphoreType.DMA,
        pltpu.SemaphoreType.DMA,
        pltpu.SemaphoreType.DMA,
        pltpu.SemaphoreType.DMA,
        pltpu.SemaphoreType.DMA,
        pltpu.SemaphoreType.DMA,
        pltpu.SemaphoreType.DMA,
    ],
    compiler_params=pltpu.CompilerParams(use_tc_tiling_on_sc=True,
                                         needs_layout_passes=False),
)
def _format_kernel(wt_hbm, out_hbm, in_v, tr0, tr1, tr2, tr3,
                   s_i0, s_i1, s_i2, s_i3, s_o0, s_o1, s_o2, s_o3):
    wid = lax.axis_index("s") * _NC + lax.axis_index("c")
    c0 = wid * _TPW
    s_in = (s_i0, s_i1, s_i2, s_i3)
    s_out = (s_o0, s_o1, s_o2, s_o3)
    tr = (tr0, tr1, tr2, tr3)

    iota = lax.iota(jnp.int32, 16)
    d_lo = iota
    d_hi = iota + 16

    def start_in(c, b):
        pltpu.async_copy(wt_hbm.at[:, pl.ds(c * 128, 128)],
                         in_v.at[b, :, pl.ds(0, 128)], s_in[b])

    def wait_in(b):
        pltpu.make_async_copy(wt_hbm.at[:, pl.ds(0, 128)],
                              in_v.at[b, :, pl.ds(0, 128)], s_in[b]).wait()

    def start_out(c, b):
        pltpu.async_copy(tr[b], out_hbm.at[pl.ds(c * 4096, 4096)], s_out[b])

    def wait_out(b):
        pltpu.make_async_copy(tr[b], out_hbm.at[pl.ds(0, 4096)],
                              s_out[b]).wait()

    def transpose_chunk(b):
        src_ref = in_v.at[b]
        dst = tr[b]

        @plsc.parallel_loop(0, 128, unroll=8)
        def body(v):
            vv = jnp.full((16,), v, jnp.int32)
            lo = plsc.load_gather(src_ref, [d_lo, vv])
            hi = plsc.load_gather(src_ref, [d_hi, vv])
            dst[pl.ds(v * 32, 16)] = lo
            dst[pl.ds(v * 32 + 16, 16)] = hi

    start_in(c0, 0)
    start_in(c0 + 1, 1)
    start_in(c0 + 2, 2)

    def quad_body(k, carry):
        for b in (0, 1, 2, 3):
            i = 4 * k + b
            c = c0 + i
            start_in(c + 3, (b + 3) % 4)
            wait_in(b)

            @pl.when(k >= 1)
            def _wout():
                wait_out(b)

            transpose_chunk(b)
            start_out(c, b)
        return carry

    lax.fori_loop(0, _TPW // 4, quad_body, 0)
    for b in range(4):
        wait_out(b)
    for b in range(3):
        wait_in(b)

    @pl.when(wid < _TXTRA)
    def _extra():
        c = _FULL_TILES - _TXTRA + wid
        start_in(c, 3)
        wait_in(3)
        transpose_chunk(3)
        start_out(c, 3)
        wait_out(3)


@functools.partial(
    pl.kernel,
    mesh=_mesh,
    out_type=jax.ShapeDtypeStruct((_NUM_ROWS, _DIM), jnp.float32),
    scratch_types=[
        pltpu.VMEM((2, _CHUNK), jnp.int32),
        pltpu.VMEM((2, _CHUNK, _DIM), jnp.float32),
        pltpu.SemaphoreType.DMA,
        pltpu.SemaphoreType.DMA,
        pltpu.SemaphoreType.DMA,
        pltpu.SemaphoreType.DMA,
        pltpu.SemaphoreType.DMA,
        pltpu.SemaphoreType.DMA,
    ],
    compiler_params=pltpu.CompilerParams(use_tc_tiling_on_sc=False),
)
def _gather_kernel(idx_hbm, table_hbm, out_hbm, idx_v, rows_v,
                   s_i0, s_i1, s_g0, s_g1, s_o0, s_o1):
    wid = lax.axis_index("s") * _NC + lax.axis_index("c")
    base = wid * _ROWS_PER_W
    s_idx = (s_i0, s_i1)
    s_gat = (s_g0, s_g1)
    s_out = (s_o0, s_o1)

    def start_idx(i):
        b = i % 2
        return pltpu.async_copy(
            idx_hbm.at[pl.ds(base + i * _CHUNK, _CHUNK)], idx_v.at[b], s_idx[b])

    def start_gather(i):
        b = i % 2
        return pltpu.async_copy(table_hbm.at[idx_v.at[b]], rows_v.at[b], s_gat[b])

    def start_out(i):
        b = i % 2
        return pltpu.async_copy(
            rows_v.at[b], out_hbm.at[pl.ds(base + i * _CHUNK, _CHUNK)], s_out[b])

    idx_h = [None] * _NCHUNK
    gat_h = [None] * _NCHUNK
    out_h = [None] * _NCHUNK

    idx_h[0] = start_idx(0)
    idx_h[1] = start_idx(1)
    for i in range(_NCHUNK):
        b = i % 2
        idx_h[i].wait()
        if i >= 2:
            out_h[i - 2].wait()
        gat_h[i] = start_gather(i)
        if i >= 1:
            gat_h[i - 1].wait()
            out_h[i - 1] = start_out(i - 1)
            if i + 1 < _NCHUNK:
                idx_h[i + 1] = start_idx(i + 1)
    gat_h[_NCHUNK - 1].wait()
    out_h[_NCHUNK - 1] = start_out(_NCHUNK - 1)
    out_h[_NCHUNK - 2].wait()
    out_h[_NCHUNK - 1].wait()


def kernel(input_, weight):
    idx = input_.reshape(-1).astype(jnp.int32)
    table_flat = _format_kernel(weight.T)
    tail = weight[_FULL_TILES * 128:, :].reshape(-1)
    table_flat = lax.dynamic_update_slice(
        table_flat, tail, (_FULL_TILES * 128 * _DIM,))
    table = table_flat.reshape(_VOCAB, _DIM)
    out = _gather_kernel(idx, table)
    return out.reshape(input_.shape + (weight.shape[1],))


# speedup vs baseline: 1.2061x; 1.0218x over previous
import functools

import jax
import jax.numpy as jnp
from jax import lax
from jax.experimental import pallas as pl
from jax.experimental.pallas import tpu as pltpu
from jax.experimental.pallas import tpu_sc as plsc

_NUM_ROWS = 4096 * 200
_DIM = 32
_VOCAB = 1000000
_FULL_TILES = _VOCAB // 128
_REM = _VOCAB - _FULL_TILES * 128

_INFO = plsc.get_sparse_core_info()
_NC = _INFO.num_cores
_NS = _INFO.num_subcores
_NW = _NC * _NS
_ROWS_PER_W = _NUM_ROWS // _NW
_CHUNK = 1600
_NCHUNK = _ROWS_PER_W // _CHUNK

_TPW = _FULL_TILES // _NW
_TXTRA = _FULL_TILES - _TPW * _NW

_mesh = plsc.VectorSubcoreMesh(core_axis_name="c", subcore_axis_name="s")


@functools.partial(
    pl.kernel,
    mesh=_mesh,
    out_type=jax.ShapeDtypeStruct((_VOCAB * _DIM,), jnp.float32),
    scratch_types=[
        pltpu.VMEM((4, 32, 129), jnp.float32),
        pltpu.VMEM((4096,), jnp.float32),
        pltpu.VMEM((4096,), jnp.float32),
        pltpu.VMEM((4096,), jnp.float32),
        pltpu.VMEM((4096,), jnp.float32),
        pltpu.SemaphoreType.DMA,
        pltpu.SemaphoreType.DMA,
        pltpu.SemaphoreType.DMA,
        pltpu.SemaphoreType.DMA,
        pltpu.SemaphoreType.DMA,
        pltpu.SemaphoreType.DMA,
        pltpu.SemaphoreType.DMA,
        pltpu.SemaphoreType.DMA,
    ],
    compiler_params=pltpu.CompilerParams(use_tc_tiling_on_sc=True,
                                         needs_layout_passes=False),
)
def _format_kernel(wt_hbm, out_hbm, in_v, tr0, tr1, tr2, tr3,
                   s_i0, s_i1, s_i2, s_i3, s_o0, s_o1, s_o2, s_o3):
    wid = lax.axis_index("s") * _NC + lax.axis_index("c")
    c0 = wid * _TPW
    s_in = (s_i0, s_i1, s_i2, s_i3)
    s_out = (s_o0, s_o1, s_o2, s_o3)
    tr = (tr0, tr1, tr2, tr3)

    iota = lax.iota(jnp.int32, 16)
    d_lo = iota
    d_hi = iota + 16

    def start_in(c, b):
        pltpu.async_copy(wt_hbm.at[:, pl.ds(c * 128, 128)],
                         in_v.at[b, :, pl.ds(0, 128)], s_in[b])

    def wait_in(b):
        pltpu.make_async_copy(wt_hbm.at[:, pl.ds(0, 128)],
                              in_v.at[b, :, pl.ds(0, 128)], s_in[b]).wait()

    def start_out(c, b):
        pltpu.async_copy(tr[b], out_hbm.at[pl.ds(c * 4096, 4096)], s_out[b])

    def wait_out(b):
        pltpu.make_async_copy(tr[b], out_hbm.at[pl.ds(0, 4096)],
                              s_out[b]).wait()

    def transpose_chunk(b):
        src_ref = in_v.at[b]
        dst = tr[b]

        @plsc.parallel_loop(0, 128, unroll=16)
        def body(v):
            vv = jnp.full((16,), v, jnp.int32)
            lo = plsc.load_gather(src_ref, [d_lo, vv])
            hi = plsc.load_gather(src_ref, [d_hi, vv])
            dst[pl.ds(v * 32, 16)] = lo
            dst[pl.ds(v * 32 + 16, 16)] = hi

    start_in(c0, 0)
    start_in(c0 + 1, 1)
    start_in(c0 + 2, 2)

    def quad_body(k, carry):
        for b in (0, 1, 2, 3):
            i = 4 * k + b
            c = c0 + i
            start_in(c + 3, (b + 3) % 4)
            wait_in(b)

            @pl.when(k >= 1)
            def _wout():
                wait_out(b)

            transpose_chunk(b)
            start_out(c, b)
        return carry

    lax.fori_loop(0, _TPW // 4, quad_body, 0)
    for b in range(4):
        wait_out(b)
    for b in range(3):
        wait_in(b)

    @pl.when(wid < _TXTRA)
    def _extra():
        c = _FULL_TILES - _TXTRA + wid
        start_in(c, 3)
        wait_in(3)
        transpose_chunk(3)
        start_out(c, 3)
        wait_out(3)


@functools.partial(
    pl.kernel,
    mesh=_mesh,
    out_type=jax.ShapeDtypeStruct((_NUM_ROWS, _DIM), jnp.float32),
    scratch_types=[
        pltpu.VMEM((2, _CHUNK), jnp.int32),
        pltpu.VMEM((2, _CHUNK, _DIM), jnp.float32),
        pltpu.SemaphoreType.DMA,
        pltpu.SemaphoreType.DMA,
        pltpu.SemaphoreType.DMA,
        pltpu.SemaphoreType.DMA,
        pltpu.SemaphoreType.DMA,
        pltpu.SemaphoreType.DMA,
    ],
    compiler_params=pltpu.CompilerParams(use_tc_tiling_on_sc=False),
)
def _gather_kernel(idx_hbm, table_hbm, out_hbm, idx_v, rows_v,
                   s_i0, s_i1, s_g0, s_g1, s_o0, s_o1):
    wid = lax.axis_index("s") * _NC + lax.axis_index("c")
    base = wid * _ROWS_PER_W
    s_idx = (s_i0, s_i1)
    s_gat = (s_g0, s_g1)
    s_out = (s_o0, s_o1)

    def start_idx(i):
        b = i % 2
        return pltpu.async_copy(
            idx_hbm.at[pl.ds(base + i * _CHUNK, _CHUNK)], idx_v.at[b], s_idx[b])

    def start_gather(i):
        b = i % 2
        return pltpu.async_copy(table_hbm.at[idx_v.at[b]], rows_v.at[b], s_gat[b])

    def start_out(i):
        b = i % 2
        return pltpu.async_copy(
            rows_v.at[b], out_hbm.at[pl.ds(base + i * _CHUNK, _CHUNK)], s_out[b])

    idx_h = [None] * _NCHUNK
    gat_h = [None] * _NCHUNK
    out_h = [None] * _NCHUNK

    idx_h[0] = start_idx(0)
    idx_h[1] = start_idx(1)
    for i in range(_NCHUNK):
        b = i % 2
        idx_h[i].wait()
        if i >= 2:
            out_h[i - 2].wait()
        gat_h[i] = start_gather(i)
        if i >= 1:
            gat_h[i - 1].wait()
            out_h[i - 1] = start_out(i - 1)
            if i + 1 < _NCHUNK:
                idx_h[i + 1] = start_idx(i + 1)
    gat_h[_NCHUNK - 1].wait()
    out_h[_NCHUNK - 1] = start_out(_NCHUNK - 1)
    out_h[_NCHUNK - 2].wait()
    out_h[_NCHUNK - 1].wait()


def kernel(input_, weight):
    idx = input_.reshape(-1).astype(jnp.int32)
    table_flat = _format_kernel(weight.T)
    tail = weight[_FULL_TILES * 128:, :].reshape(-1)
    table_flat = lax.dynamic_update_slice(
        table_flat, tail, (_FULL_TILES * 128 * _DIM,))
    table = table_flat.reshape(_VOCAB, _DIM)
    out = _gather_kernel(idx, table)
    return out.reshape(input_.shape + (weight.shape[1],))


# speedup vs baseline: 1.2084x; 1.0019x over previous
import functools

import jax
import jax.numpy as jnp
from jax import lax
from jax.experimental import pallas as pl
from jax.experimental.pallas import tpu as pltpu
from jax.experimental.pallas import tpu_sc as plsc

_NUM_ROWS = 4096 * 200
_DIM = 32
_VOCAB = 1000000
_FULL_TILES = _VOCAB // 128
_REM = _VOCAB - _FULL_TILES * 128

_INFO = plsc.get_sparse_core_info()
_NC = _INFO.num_cores
_NS = _INFO.num_subcores
_NW = _NC * _NS
_ROWS_PER_W = _NUM_ROWS // _NW
_CHUNK = 1600
_NCHUNK = _ROWS_PER_W // _CHUNK

_TPW = _FULL_TILES // _NW
_TXTRA = _FULL_TILES - _TPW * _NW

_mesh = plsc.VectorSubcoreMesh(core_axis_name="c", subcore_axis_name="s")


@functools.partial(
    pl.kernel,
    mesh=_mesh,
    out_type=jax.ShapeDtypeStruct((_VOCAB * _DIM,), jnp.float32),
    scratch_types=[
        pltpu.VMEM((4, 32, 129), jnp.float32),
        pltpu.VMEM((4096,), jnp.float32),
        pltpu.VMEM((4096,), jnp.float32),
        pltpu.VMEM((4096,), jnp.float32),
        pltpu.VMEM((4096,), jnp.float32),
        pltpu.SemaphoreType.DMA,
        pltpu.SemaphoreType.DMA,
        pltpu.SemaphoreType.DMA,
        pltpu.SemaphoreType.DMA,
        pltpu.SemaphoreType.DMA,
        pltpu.SemaphoreType.DMA,
        pltpu.SemaphoreType.DMA,
        pltpu.SemaphoreType.DMA,
    ],
    compiler_params=pltpu.CompilerParams(use_tc_tiling_on_sc=True,
                                         needs_layout_passes=False),
)
def _format_kernel(wt_hbm, out_hbm, in_v, tr0, tr1, tr2, tr3,
                   s_i0, s_i1, s_i2, s_i3, s_o0, s_o1, s_o2, s_o3):
    wid = lax.axis_index("s") * _NC + lax.axis_index("c")
    c0 = wid * _TPW
    s_in = (s_i0, s_i1, s_i2, s_i3)
    s_out = (s_o0, s_o1, s_o2, s_o3)
    tr = (tr0, tr1, tr2, tr3)

    iota = lax.iota(jnp.int32, 16)
    d_lo = iota
    d_hi = iota + 16

    def start_in(c, b):
        pltpu.async_copy(wt_hbm.at[:, pl.ds(c * 128, 128)],
                         in_v.at[b, :, pl.ds(0, 128)], s_in[b])

    def wait_in(b):
        pltpu.make_async_copy(wt_hbm.at[:, pl.ds(0, 128)],
                              in_v.at[b, :, pl.ds(0, 128)], s_in[b]).wait()

    def start_out(c, b):
        pltpu.async_copy(tr[b], out_hbm.at[pl.ds(c * 4096, 4096)], s_out[b])

    def wait_out(b):
        pltpu.make_async_copy(tr[b], out_hbm.at[pl.ds(0, 4096)],
                              s_out[b]).wait()

    def transpose_chunk(b):
        src_ref = in_v.at[b]
        dst = tr[b]

        @plsc.parallel_loop(0, 128, unroll=32)
        def body(v):
            vv = jnp.full((16,), v, jnp.int32)
            lo = plsc.load_gather(src_ref, [d_lo, vv])
            hi = plsc.load_gather(src_ref, [d_hi, vv])
            dst[pl.ds(v * 32, 16)] = lo
            dst[pl.ds(v * 32 + 16, 16)] = hi

    start_in(c0, 0)
    start_in(c0 + 1, 1)
    start_in(c0 + 2, 2)

    def quad_body(k, carry):
        for b in (0, 1, 2, 3):
            i = 4 * k + b
            c = c0 + i
            start_in(c + 3, (b + 3) % 4)
            wait_in(b)

            @pl.when(k >= 1)
            def _wout():
                wait_out(b)

            transpose_chunk(b)
            start_out(c, b)
        return carry

    lax.fori_loop(0, _TPW // 4, quad_body, 0)
    for b in range(4):
        wait_out(b)
    for b in range(3):
        wait_in(b)

    @pl.when(wid < _TXTRA)
    def _extra():
        c = _FULL_TILES - _TXTRA + wid
        start_in(c, 3)
        wait_in(3)
        transpose_chunk(3)
        start_out(c, 3)
        wait_out(3)


@functools.partial(
    pl.kernel,
    mesh=_mesh,
    out_type=jax.ShapeDtypeStruct((_NUM_ROWS, _DIM), jnp.float32),
    scratch_types=[
        pltpu.VMEM((2, _CHUNK), jnp.int32),
        pltpu.VMEM((2, _CHUNK, _DIM), jnp.float32),
        pltpu.SemaphoreType.DMA,
        pltpu.SemaphoreType.DMA,
        pltpu.SemaphoreType.DMA,
        pltpu.SemaphoreType.DMA,
        pltpu.SemaphoreType.DMA,
        pltpu.SemaphoreType.DMA,
    ],
    compiler_params=pltpu.CompilerParams(use_tc_tiling_on_sc=False),
)
def _gather_kernel(idx_hbm, table_hbm, out_hbm, idx_v, rows_v,
                   s_i0, s_i1, s_g0, s_g1, s_o0, s_o1):
    wid = lax.axis_index("s") * _NC + lax.axis_index("c")
    base = wid * _ROWS_PER_W
    s_idx = (s_i0, s_i1)
    s_gat = (s_g0, s_g1)
    s_out = (s_o0, s_o1)

    def start_idx(i):
        b = i % 2
        return pltpu.async_copy(
            idx_hbm.at[pl.ds(base + i * _CHUNK, _CHUNK)], idx_v.at[b], s_idx[b])

    def start_gather(i):
        b = i % 2
        return pltpu.async_copy(table_hbm.at[idx_v.at[b]], rows_v.at[b], s_gat[b])

    def start_out(i):
        b = i % 2
        return pltpu.async_copy(
            rows_v.at[b], out_hbm.at[pl.ds(base + i * _CHUNK, _CHUNK)], s_out[b])

    idx_h = [None] * _NCHUNK
    gat_h = [None] * _NCHUNK
    out_h = [None] * _NCHUNK

    idx_h[0] = start_idx(0)
    idx_h[1] = start_idx(1)
    for i in range(_NCHUNK):
        b = i % 2
        idx_h[i].wait()
        if i >= 2:
            out_h[i - 2].wait()
        gat_h[i] = start_gather(i)
        if i >= 1:
            gat_h[i - 1].wait()
            out_h[i - 1] = start_out(i - 1)
            if i + 1 < _NCHUNK:
                idx_h[i + 1] = start_idx(i + 1)
    gat_h[_NCHUNK - 1].wait()
    out_h[_NCHUNK - 1] = start_out(_NCHUNK - 1)
    out_h[_NCHUNK - 2].wait()
    out_h[_NCHUNK - 1].wait()


def kernel(input_, weight):
    idx = input_.reshape(-1).astype(jnp.int32)
    table_flat = _format_kernel(weight.T)
    tail = weight[_FULL_TILES * 128:, :].reshape(-1)
    table_flat = lax.dynamic_update_slice(
        table_flat, tail, (_FULL_TILES * 128 * _DIM,))
    table = table_flat.reshape(_VOCAB, _DIM)
    out = _gather_kernel(idx, table)
    return out.reshape(input_.shape + (weight.shape[1],))
